# baseline (speedup 1.0000x reference)
import functools
import jax
import jax.numpy as jnp
from jax import lax
from jax.experimental import pallas as pl
from jax.experimental.pallas import tpu as pltpu
from jax.experimental.pallas import tpu_sc as plsc

N = 10000
E = 160000
NUM_ELEM = 10
F = 128
NB = 8
RMAX = 5.0
AVG = 16.0

BE = 1024
BN = 1000

_EPAD = 163840
_P = 2
_ESL = _EPAD // _P
_ZROWS = 624
_AROWS = N + 16

_S3 = 3.0 ** 0.5
_S15 = 15.0 ** 0.5
_S5H = (5.0 ** 0.5) / 2.0
_S15H = _S15 / 2.0


def _edge_body(gs_ref, gd_ref, w1_ref, w2_ref, w3_ref, we_ref, out_ref):
    g_s = gs_ref[...]
    g_d = gd_ref[...]
    d = g_s - g_d
    x, y, z = d[:, 0:1], d[:, 1:2], d[:, 2:3]
    r2 = x * x + y * y + z * z + 1e-9
    rinv = lax.rsqrt(r2)
    r = r2 * rinv
    ux, uy, uz = x * rinv, y * rinv, z * rinv

    nvec = ((jnp.arange(NB, dtype=jnp.int32).astype(jnp.float32) + 1.0)
            * (jnp.pi / RMAX))[None, :]
    xr = r * (1.0 / RMAX)
    x5 = xr * xr * xr * xr * xr
    cut = (1.0 - 21.0 * x5 + 35.0 * x5 * xr - 15.0 * x5 * xr * xr)
    cut = jnp.where(xr < 1.0, cut, 0.0)
    scale = ((2.0 / RMAX) ** 0.5) * rinv * cut
    rb = jnp.sin(r * nvec) * scale

    h = rb @ w1_ref[...]
    h = h * jax.nn.sigmoid(h)
    h = h @ w2_ref[...]
    h = h * jax.nn.sigmoid(h)
    rw = h @ w3_ref[...]

    zbits = lax.bitcast_convert_type(g_s[:, 3:4], jnp.int32)
    oh = (zbits == jnp.arange(NUM_ELEM, dtype=jnp.int32)[None, :]).astype(jnp.float32)
    nf = oh @ we_ref[...]
    hs = nf * rw

    out_ref[0] = hs
    out_ref[1] = hs * (_S3 * uy)
    out_ref[2] = hs * (_S3 * uz)
    out_ref[3] = hs * (_S3 * ux)
    out_ref[4] = hs * (_S15 * ux * uy)
    out_ref[5] = hs * (_S15 * uy * uz)
    out_ref[6] = hs * (_S5H * (3.0 * uz * uz - 1.0))
    out_ref[7] = hs * (_S15 * ux * uz)
    out_ref[8] = hs * (_S15H * (ux * ux - uy * uy))


def _edge_messages(g_src, g_dst, W1, W2, W3, W_embed):
    ne = g_src.shape[0]
    return pl.pallas_call(
        _edge_body,
        grid=(ne // BE,),
        in_specs=[
            pl.BlockSpec((BE, F), lambda i: (i, 0)),
            pl.BlockSpec((BE, F), lambda i: (i, 0)),
            pl.BlockSpec((NB, 64), lambda i: (0, 0)),
            pl.BlockSpec((64, 64), lambda i: (0, 0)),
            pl.BlockSpec((64, F), lambda i: (0, 0)),
            pl.BlockSpec((NUM_ELEM, F), lambda i: (0, 0)),
        ],
        out_specs=pl.BlockSpec((9, BE, F), lambda i: (0, i, 0)),
        out_shape=jax.ShapeDtypeStruct((9, ne, F), jnp.float32),
    )(g_src, g_dst, W1, W2, W3, W_embed)


def _make_sc_gather(ne):
    etile = ne // 32
    ncht = etile // 128

    def body(tbl_hbm, idx2_hbm, gsrc_hbm, gdst_hbm, idx_v, buf_a, buf_b,
             sem_a, sem_b):
        c = lax.axis_index("c")
        s = lax.axis_index("s")
        w = c * 16 + s
        ebase = w * etile
        crow = w * ncht

        bufs = (buf_a, buf_b)
        sems = (sem_a, sem_b)
        outs = (gsrc_hbm, gdst_hbm)

        del crow
        pltpu.sync_copy(idx2_hbm.at[0, w], idx_v.at[pl.ds(0, ncht)])
        pltpu.sync_copy(idx2_hbm.at[1, w], idx_v.at[pl.ds(ncht, ncht)])

        def gat(t, g, buf, sem):
            pltpu.async_copy(tbl_hbm.at[idx_v.at[t * ncht + g]], buf, sem)

        def wait_gat(t, g, buf, sem):
            pltpu.make_async_copy(tbl_hbm.at[idx_v.at[t * ncht + g]], buf, sem).wait()

        def put(t, g, buf, sem):
            pltpu.async_copy(buf, outs[t].at[pl.ds(ebase + g * 128, 128)], sem)

        def wait_put(t, g, buf, sem):
            pltpu.make_async_copy(buf, outs[t].at[pl.ds(ebase + g * 128, 128)],
                                  sem).wait()

        for t in range(2):
            gat(t, 0, buf_a, sem_a)

            def grp(go, carry):
                for par in range(2):
                    g = go * 2 + par
                    buf, sem = bufs[par], sems[par]
                    obuf, osem = bufs[1 - par], sems[1 - par]
                    wait_gat(t, g, buf, sem)

                    @pl.when(g >= 1)
                    def _():
                        wait_put(t, g - 1, obuf, osem)

                    @pl.when(g + 1 < ncht)
                    def _():
                        gat(t, g + 1, obuf, osem)

                    put(t, g, buf, sem)
                return carry

            lax.fori_loop(0, ncht // 2, grp, 0)
            wait_put(t, ncht - 1, bufs[1], sems[1])

    return pl.kernel(
        body,
        mesh=plsc.VectorSubcoreMesh(core_axis_name="c", subcore_axis_name="s"),
        out_type=(jax.ShapeDtypeStruct((ne, F), jnp.float32),
                  jax.ShapeDtypeStruct((ne, F), jnp.float32)),
        name="sc_gather",
        scratch_types=[
            pltpu.VMEM((2 * ncht, 128), jnp.int32),
            pltpu.VMEM((128, F), jnp.float32),
            pltpu.VMEM((128, F), jnp.float32),
            pltpu.SemaphoreType.DMA, pltpu.SemaphoreType.DMA,
        ],
    )


def _make_sc_scatter(ne):
    ecore = ne // 2
    etile = ecore // 16
    ncht = etile // 128

    def body(msg_hbm, dst2_hbm, part_hbm, idx_v, buf_a, buf_b, zbuf, accum,
             sem_la, sem_lb, sem_sa, sem_sb):
        c = lax.axis_index("c")
        s = lax.axis_index("s")
        ebase = c * ecore + s * etile
        w = c * 16 + s
        zstart = _ZROWS * s

        bufs = (buf_a, buf_b)
        ldse = (sem_la, sem_lb)
        scse = (sem_sa, sem_sb)

        pltpu.sync_copy(dst2_hbm.at[w], idx_v)

        def zrow(i, carry):
            for j8 in range(8):
                zbuf[i, pl.ds(j8 * 16, 16)] = jnp.zeros((16,), jnp.float32)
            return carry

        lax.fori_loop(0, 64, zrow, 0)

        def load_group(k, g, buf, sem):
            pltpu.async_copy(msg_hbm.at[k, pl.ds(ebase + g * 128, 128)], buf, sem)

        def wait_load(k, g, buf, sem):
            pltpu.make_async_copy(
                msg_hbm.at[k, pl.ds(ebase + g * 128, 128)], buf, sem).wait()

        def scat(g, buf, sem):
            pltpu.async_copy(buf, accum.at[idx_v.at[g]], sem, add=True)

        def wait_scat(g, buf, sem):
            pltpu.make_async_copy(buf, accum.at[idx_v.at[g]], sem).wait()

        def zero_issue(sem):
            for rep in range(9):
                pltpu.async_copy(zbuf, accum.at[pl.ds(zstart + rep * 64, 64)], sem)
            pltpu.async_copy(zbuf.at[pl.ds(0, 48)],
                             accum.at[pl.ds(zstart + 576, 48)], sem)

            @pl.when(s == 15)
            def _():
                pltpu.async_copy(zbuf.at[pl.ds(0, 32)],
                                 accum.at[pl.ds(16 * _ZROWS, 32)], sem)

        def zero_wait(sem):
            for rep in range(9):
                pltpu.make_async_copy(
                    zbuf, accum.at[pl.ds(zstart + rep * 64, 64)], sem).wait()
            pltpu.make_async_copy(zbuf.at[pl.ds(0, 48)],
                                  accum.at[pl.ds(zstart + 576, 48)], sem).wait()

            @pl.when(s == 15)
            def _():
                pltpu.make_async_copy(zbuf.at[pl.ds(0, 32)],
                                      accum.at[pl.ds(16 * _ZROWS, 32)], sem).wait()

        for k in range(9):
            zero_issue(sem_la)
            zero_wait(sem_la)
            plsc.subcore_barrier()

            load_group(k, 0, buf_a, sem_la)

            def grp(go, carry):
                for par in range(2):
                    g = go * 2 + par
                    buf, ss = bufs[par], scse[par]
                    obuf, ols, oss = bufs[1 - par], ldse[1 - par], scse[1 - par]
                    wait_load(k, g, buf, ldse[par])

                    @pl.when(g >= 1)
                    def _():
                        wait_scat(g - 1, obuf, oss)

                    @pl.when(g + 1 < ncht)
                    def _():
                        load_group(k, g + 1, obuf, ols)

                    scat(g, buf, ss)
                return carry

            lax.fori_loop(0, ncht // 2, grp, 0)
            wait_scat(ncht - 1, bufs[1], scse[1])
            plsc.subcore_barrier()

            pltpu.sync_copy(accum.at[pl.ds(zstart, _ZROWS)],
                            part_hbm.at[c, k, pl.ds(zstart, _ZROWS)])

            @pl.when(s == 15)
            def _():
                pltpu.sync_copy(accum.at[pl.ds(16 * _ZROWS, 16)],
                                part_hbm.at[c, k, pl.ds(16 * _ZROWS, 16)])

            plsc.subcore_barrier()

    return pl.kernel(
        body,
        mesh=plsc.VectorSubcoreMesh(core_axis_name="c", subcore_axis_name="s"),
        out_type=jax.ShapeDtypeStruct((2, 9, N, F), jnp.float32),
        scratch_types=[
            pltpu.VMEM((ncht, 128), jnp.int32),
            pltpu.VMEM((128, F), jnp.float32),
            pltpu.VMEM((128, F), jnp.float32),
            pltpu.VMEM((64, F), jnp.float32),
            pltpu.VMEM_SHARED((_AROWS, F), jnp.float32),
            pltpu.SemaphoreType.DMA, pltpu.SemaphoreType.DMA,
            pltpu.SemaphoreType.DMA, pltpu.SemaphoreType.DMA,
        ],
    )


def _node_body(p0_ref, p1_ref, z_ref, wp_ref, wr_ref, ae_ref, out_ref):
    a = (jnp.sum(p0_ref[...], axis=0)
         + jnp.sum(p1_ref[...], axis=0)) * (1.0 / AVG)
    inv0 = a[0]
    inv1 = a[1] * a[1] + a[2] * a[2] + a[3] * a[3]
    inv2 = a[4] * a[4] + a[5] * a[5] + a[6] * a[6] + a[7] * a[7] + a[8] * a[8]
    inv = jnp.concatenate([inv0, inv1, inv2], axis=-1)
    h = inv @ wp_ref[...]
    h = h * jax.nn.sigmoid(h)
    node_e = jnp.sum(h * wr_ref[...], axis=1, keepdims=True)
    oh = (z_ref[...] == jnp.arange(NUM_ELEM, dtype=jnp.int32)[None, :]).astype(jnp.float32)
    e0 = jnp.sum(oh * ae_ref[...], axis=1, keepdims=True)
    out_ref[...] = e0 + node_e


def _node_energy(p0, p1, node_z, W_prod, W_read, atomic_energies):
    out = pl.pallas_call(
        _node_body,
        grid=(N // BN,),
        in_specs=[
            pl.BlockSpec((2, 9, BN, F), lambda i: (0, 0, i, 0)),
            pl.BlockSpec((2, 9, BN, F), lambda i: (0, 0, i, 0)),
            pl.BlockSpec((BN, 1), lambda i: (i, 0)),
            pl.BlockSpec((3 * F, F), lambda i: (0, 0)),
            pl.BlockSpec((1, F), lambda i: (0, 0)),
            pl.BlockSpec((1, NUM_ELEM), lambda i: (0, 0)),
        ],
        out_specs=pl.BlockSpec((BN, 1), lambda i: (i, 0)),
        out_shape=jax.ShapeDtypeStruct((N, 1), jnp.float32),
    )(p0, p1, node_z[:, None], W_prod, W_read.T, atomic_energies[None, :])
    return out[:, 0]


def kernel(positions, node_z, edge_index, W_embed, W1, W2, W3, W_prod, W_read, atomic_energies):
    node_z = node_z.astype(jnp.int32)
    src, dst = edge_index[0], edge_index[1]
    tbl = jnp.zeros((N, F), jnp.float32)
    tbl = tbl.at[:, 0:3].set(positions)
    tbl = tbl.at[:, 3].set(lax.bitcast_convert_type(node_z, jnp.float32))

    pad = _EPAD - E
    zpad = jnp.zeros((pad,), jnp.int32)
    ncht = _ESL // 32 // 128
    idx2 = jnp.stack([jnp.concatenate([src.astype(jnp.int32), zpad]),
                      jnp.concatenate([dst.astype(jnp.int32), zpad])])
    idx2 = idx2.reshape(2, _P, 32, ncht, 128)
    dst2 = jnp.concatenate([dst.astype(jnp.int32),
                            jnp.full((pad,), N, jnp.int32)])
    dst2 = dst2.reshape(_P, 32, ncht, 128)

    sc_gather = _make_sc_gather(_ESL)
    sc_scatter = _make_sc_scatter(_ESL)

    parts = []
    for p in range(_P):
        g_src, g_dst = sc_gather(tbl, idx2[:, p])
        msg = _edge_messages(g_src, g_dst, W1, W2, W3, W_embed)
        parts.append(sc_scatter(msg, dst2[p]))

    return _node_energy(parts[0], parts[1], node_z, W_prod, W_read,
                        atomic_energies)


# speedup vs baseline: 1.2889x; 1.2889x over previous
import functools
import jax
import jax.numpy as jnp
from jax import lax
from jax.experimental import pallas as pl
from jax.experimental.pallas import tpu as pltpu
from jax.experimental.pallas import tpu_sc as plsc

N = 10000
E = 160000
NUM_ELEM = 10
F = 128
NB = 8
RMAX = 5.0
AVG = 16.0

BE = 1024
BN = 1000

_EPAD = 163840
_P = 2
_ESL = _EPAD // _P
_ZROWS = 624
_AROWS = N + 16

_S3 = 3.0 ** 0.5
_S15 = 15.0 ** 0.5
_S5H = (5.0 ** 0.5) / 2.0
_S15H = _S15 / 2.0


def _edge_body(gs_ref, gd_ref, w1_ref, w2_ref, w3_ref, we_ref, out_ref):
    g_s = gs_ref[...]
    g_d = gd_ref[...]
    d = g_s - g_d
    x, y, z = d[:, 0:1], d[:, 1:2], d[:, 2:3]
    r2 = x * x + y * y + z * z + 1e-9
    rinv = lax.rsqrt(r2)
    r = r2 * rinv
    ux, uy, uz = x * rinv, y * rinv, z * rinv

    nvec = ((jnp.arange(NB, dtype=jnp.int32).astype(jnp.float32) + 1.0)
            * (jnp.pi / RMAX))[None, :]
    xr = r * (1.0 / RMAX)
    x5 = xr * xr * xr * xr * xr
    cut = (1.0 - 21.0 * x5 + 35.0 * x5 * xr - 15.0 * x5 * xr * xr)
    cut = jnp.where(xr < 1.0, cut, 0.0)
    scale = ((2.0 / RMAX) ** 0.5) * rinv * cut
    rb = jnp.sin(r * nvec) * scale

    h = rb @ w1_ref[...]
    h = h * jax.nn.sigmoid(h)
    h = h @ w2_ref[...]
    h = h * jax.nn.sigmoid(h)
    rw = h @ w3_ref[...]

    zbits = lax.bitcast_convert_type(g_s[:, 3:4], jnp.int32)
    oh = (zbits == jnp.arange(NUM_ELEM, dtype=jnp.int32)[None, :]).astype(jnp.float32)
    nf = oh @ we_ref[...]
    hs = nf * rw

    out_ref[0] = hs
    out_ref[1] = hs * (_S3 * uy)
    out_ref[2] = hs * (_S3 * uz)
    out_ref[3] = hs * (_S3 * ux)
    out_ref[4] = hs * (_S15 * ux * uy)
    out_ref[5] = hs * (_S15 * uy * uz)
    out_ref[6] = hs * (_S5H * (3.0 * uz * uz - 1.0))
    out_ref[7] = hs * (_S15 * ux * uz)
    out_ref[8] = hs * (_S15H * (ux * ux - uy * uy))


def _edge_messages(g_src, g_dst, W1, W2, W3, W_embed):
    ne = g_src.shape[0]
    return pl.pallas_call(
        _edge_body,
        grid=(ne // BE,),
        in_specs=[
            pl.BlockSpec((BE, 16), lambda i: (i, 0)),
            pl.BlockSpec((BE, 16), lambda i: (i, 0)),
            pl.BlockSpec((NB, 64), lambda i: (0, 0)),
            pl.BlockSpec((64, 64), lambda i: (0, 0)),
            pl.BlockSpec((64, F), lambda i: (0, 0)),
            pl.BlockSpec((NUM_ELEM, F), lambda i: (0, 0)),
        ],
        out_specs=pl.BlockSpec((9, BE, F), lambda i: (0, i, 0)),
        out_shape=jax.ShapeDtypeStruct((9, ne, F), jnp.float32),
    )(g_src, g_dst, W1, W2, W3, W_embed)


def _make_sc_gather(ne):
    etile = ne // 32
    ncht = etile // 128

    def body(tbl_hbm, idx2_hbm, gsrc_hbm, gdst_hbm, idx_v, buf_a, buf_b,
             sem_a, sem_b):
        c = lax.axis_index("c")
        s = lax.axis_index("s")
        w = c * 16 + s
        ebase = w * etile
        crow = w * ncht

        bufs = (buf_a, buf_b)
        sems = (sem_a, sem_b)
        outs = (gsrc_hbm, gdst_hbm)

        del crow
        pltpu.sync_copy(idx2_hbm.at[0, w], idx_v.at[pl.ds(0, ncht)])
        pltpu.sync_copy(idx2_hbm.at[1, w], idx_v.at[pl.ds(ncht, ncht)])

        def gat(t, g, buf, sem):
            pltpu.async_copy(tbl_hbm.at[idx_v.at[t * ncht + g]], buf, sem)

        def wait_gat(t, g, buf, sem):
            pltpu.make_async_copy(tbl_hbm.at[idx_v.at[t * ncht + g]], buf, sem).wait()

        def put(t, g, buf, sem):
            pltpu.async_copy(buf, outs[t].at[pl.ds(ebase + g * 128, 128)], sem)

        def wait_put(t, g, buf, sem):
            pltpu.make_async_copy(buf, outs[t].at[pl.ds(ebase + g * 128, 128)],
                                  sem).wait()

        for t in range(2):
            gat(t, 0, buf_a, sem_a)

            def grp(go, carry):
                for par in range(2):
                    g = go * 2 + par
                    buf, sem = bufs[par], sems[par]
                    obuf, osem = bufs[1 - par], sems[1 - par]
                    wait_gat(t, g, buf, sem)

                    @pl.when(g >= 1)
                    def _():
                        wait_put(t, g - 1, obuf, osem)

                    @pl.when(g + 1 < ncht)
                    def _():
                        gat(t, g + 1, obuf, osem)

                    put(t, g, buf, sem)
                return carry

            lax.fori_loop(0, ncht // 2, grp, 0)
            wait_put(t, ncht - 1, bufs[1], sems[1])

    return pl.kernel(
        body,
        mesh=plsc.VectorSubcoreMesh(core_axis_name="c", subcore_axis_name="s"),
        out_type=(jax.ShapeDtypeStruct((ne, 16), jnp.float32),
                  jax.ShapeDtypeStruct((ne, 16), jnp.float32)),
        name="sc_gather",
        scratch_types=[
            pltpu.VMEM((2 * ncht, 128), jnp.int32),
            pltpu.VMEM((128, 16), jnp.float32),
            pltpu.VMEM((128, 16), jnp.float32),
            pltpu.SemaphoreType.DMA, pltpu.SemaphoreType.DMA,
        ],
        compiler_params=pltpu.CompilerParams(use_tc_tiling_on_sc=False),
    )


def _make_sc_scatter(ne):
    ecore = ne // 2
    etile = ecore // 16
    ncht = etile // 128

    def body(msg_hbm, dst2_hbm, part_hbm, idx_v, buf_a, buf_b, zbuf, accum,
             sem_la, sem_lb, sem_sa, sem_sb):
        c = lax.axis_index("c")
        s = lax.axis_index("s")
        ebase = c * ecore + s * etile
        w = c * 16 + s
        zstart = _ZROWS * s

        bufs = (buf_a, buf_b)
        ldse = (sem_la, sem_lb)
        scse = (sem_sa, sem_sb)

        pltpu.sync_copy(dst2_hbm.at[w], idx_v)

        def zrow(i, carry):
            for j8 in range(8):
                zbuf[i, pl.ds(j8 * 16, 16)] = jnp.zeros((16,), jnp.float32)
            return carry

        lax.fori_loop(0, 64, zrow, 0)

        def load_group(k, g, buf, sem):
            pltpu.async_copy(msg_hbm.at[k, pl.ds(ebase + g * 128, 128)], buf, sem)

        def wait_load(k, g, buf, sem):
            pltpu.make_async_copy(
                msg_hbm.at[k, pl.ds(ebase + g * 128, 128)], buf, sem).wait()

        def scat(g, buf, sem):
            pltpu.async_copy(buf, accum.at[idx_v.at[g]], sem, add=True)

        def wait_scat(g, buf, sem):
            pltpu.make_async_copy(buf, accum.at[idx_v.at[g]], sem).wait()

        def zero_issue(sem):
            for rep in range(9):
                pltpu.async_copy(zbuf, accum.at[pl.ds(zstart + rep * 64, 64)], sem)
            pltpu.async_copy(zbuf.at[pl.ds(0, 48)],
                             accum.at[pl.ds(zstart + 576, 48)], sem)

            @pl.when(s == 15)
            def _():
                pltpu.async_copy(zbuf.at[pl.ds(0, 32)],
                                 accum.at[pl.ds(16 * _ZROWS, 32)], sem)

        def zero_wait(sem):
            for rep in range(9):
                pltpu.make_async_copy(
                    zbuf, accum.at[pl.ds(zstart + rep * 64, 64)], sem).wait()
            pltpu.make_async_copy(zbuf.at[pl.ds(0, 48)],
                                  accum.at[pl.ds(zstart + 576, 48)], sem).wait()

            @pl.when(s == 15)
            def _():
                pltpu.make_async_copy(zbuf.at[pl.ds(0, 32)],
                                      accum.at[pl.ds(16 * _ZROWS, 32)], sem).wait()

        for k in range(9):
            zero_issue(sem_la)
            zero_wait(sem_la)
            plsc.subcore_barrier()

            load_group(k, 0, buf_a, sem_la)

            def grp(go, carry):
                for par in range(2):
                    g = go * 2 + par
                    buf, ss = bufs[par], scse[par]
                    obuf, ols, oss = bufs[1 - par], ldse[1 - par], scse[1 - par]
                    wait_load(k, g, buf, ldse[par])

                    @pl.when(g >= 1)
                    def _():
                        wait_scat(g - 1, obuf, oss)

                    @pl.when(g + 1 < ncht)
                    def _():
                        load_group(k, g + 1, obuf, ols)

                    scat(g, buf, ss)
                return carry

            lax.fori_loop(0, ncht // 2, grp, 0)
            wait_scat(ncht - 1, bufs[1], scse[1])
            plsc.subcore_barrier()

            pltpu.sync_copy(accum.at[pl.ds(zstart, _ZROWS)],
                            part_hbm.at[c, k, pl.ds(zstart, _ZROWS)])

            @pl.when(s == 15)
            def _():
                pltpu.sync_copy(accum.at[pl.ds(16 * _ZROWS, 16)],
                                part_hbm.at[c, k, pl.ds(16 * _ZROWS, 16)])

            plsc.subcore_barrier()

    return pl.kernel(
        body,
        mesh=plsc.VectorSubcoreMesh(core_axis_name="c", subcore_axis_name="s"),
        out_type=jax.ShapeDtypeStruct((2, 9, N, F), jnp.float32),
        scratch_types=[
            pltpu.VMEM((ncht, 128), jnp.int32),
            pltpu.VMEM((128, F), jnp.float32),
            pltpu.VMEM((128, F), jnp.float32),
            pltpu.VMEM((64, F), jnp.float32),
            pltpu.VMEM_SHARED((_AROWS, F), jnp.float32),
            pltpu.SemaphoreType.DMA, pltpu.SemaphoreType.DMA,
            pltpu.SemaphoreType.DMA, pltpu.SemaphoreType.DMA,
        ],
    )


def _node_body(p0_ref, p1_ref, z_ref, wp_ref, wr_ref, ae_ref, out_ref):
    a = (jnp.sum(p0_ref[...], axis=0)
         + jnp.sum(p1_ref[...], axis=0)) * (1.0 / AVG)
    inv0 = a[0]
    inv1 = a[1] * a[1] + a[2] * a[2] + a[3] * a[3]
    inv2 = a[4] * a[4] + a[5] * a[5] + a[6] * a[6] + a[7] * a[7] + a[8] * a[8]
    inv = jnp.concatenate([inv0, inv1, inv2], axis=-1)
    h = inv @ wp_ref[...]
    h = h * jax.nn.sigmoid(h)
    node_e = jnp.sum(h * wr_ref[...], axis=1, keepdims=True)
    oh = (z_ref[...] == jnp.arange(NUM_ELEM, dtype=jnp.int32)[None, :]).astype(jnp.float32)
    e0 = jnp.sum(oh * ae_ref[...], axis=1, keepdims=True)
    out_ref[...] = e0 + node_e


def _node_energy(p0, p1, node_z, W_prod, W_read, atomic_energies):
    out = pl.pallas_call(
        _node_body,
        grid=(N // BN,),
        in_specs=[
            pl.BlockSpec((2, 9, BN, F), lambda i: (0, 0, i, 0)),
            pl.BlockSpec((2, 9, BN, F), lambda i: (0, 0, i, 0)),
            pl.BlockSpec((BN, 1), lambda i: (i, 0)),
            pl.BlockSpec((3 * F, F), lambda i: (0, 0)),
            pl.BlockSpec((1, F), lambda i: (0, 0)),
            pl.BlockSpec((1, NUM_ELEM), lambda i: (0, 0)),
        ],
        out_specs=pl.BlockSpec((BN, 1), lambda i: (i, 0)),
        out_shape=jax.ShapeDtypeStruct((N, 1), jnp.float32),
    )(p0, p1, node_z[:, None], W_prod, W_read.T, atomic_energies[None, :])
    return out[:, 0]


def kernel(positions, node_z, edge_index, W_embed, W1, W2, W3, W_prod, W_read, atomic_energies):
    node_z = node_z.astype(jnp.int32)
    src, dst = edge_index[0], edge_index[1]
    tbl = jnp.zeros((N, 16), jnp.float32)
    tbl = tbl.at[:, 0:3].set(positions)
    tbl = tbl.at[:, 3].set(lax.bitcast_convert_type(node_z, jnp.float32))

    pad = _EPAD - E
    zpad = jnp.zeros((pad,), jnp.int32)
    ncht = _ESL // 32 // 128
    idx2 = jnp.stack([jnp.concatenate([src.astype(jnp.int32), zpad]),
                      jnp.concatenate([dst.astype(jnp.int32), zpad])])
    idx2 = idx2.reshape(2, _P, 32, ncht, 128)
    dst2 = jnp.concatenate([dst.astype(jnp.int32),
                            jnp.full((pad,), N, jnp.int32)])
    dst2 = dst2.reshape(_P, 32, ncht, 128)

    sc_gather = _make_sc_gather(_ESL)
    sc_scatter = _make_sc_scatter(_ESL)

    parts = []
    for p in range(_P):
        g_src, g_dst = sc_gather(tbl, idx2[:, p])
        msg = _edge_messages(g_src, g_dst, W1, W2, W3, W_embed)
        parts.append(sc_scatter(msg, dst2[p]))

    return _node_energy(parts[0], parts[1], node_z, W_prod, W_read,
                        atomic_energies)
